# Initial kernel scaffold; baseline (speedup 1.0000x reference)
import functools

import jax
import jax.numpy as jnp
from jax import lax
from jax.experimental import pallas as pl
from jax.experimental.pallas import tpu as pltpu
from jax.experimental.pallas import tpu_sc as plsc

N = 10000
D = 128
E = 320000
NC = 2
NS = 16
W = NC * NS
K = 128
C = -(-E // (W * K))
EPAD = W * C * K
STRIPE = N // NS
BB = 125


def _accum_body(x_hbm, src_hbm, dst_hbm, w_hbm, part_hbm,
                acc, src_v, dst_v, w_v, rows_v, sem):
    cid = lax.axis_index("c")
    sid = lax.axis_index("s")
    wid = cid * NS + sid

    iota16 = lax.iota(jnp.int32, 16)
    ones16 = jnp.ones((16,), jnp.int32)
    zeros16 = jnp.zeros((16,), jnp.float32)

    def _zrow(r, _):
        for j in range(D // 16):
            rows_v[r, pl.ds(16 * j, 16)] = zeros16
        return 0
    lax.fori_loop(0, K, _zrow, 0)
    base = sid * STRIPE
    for i in range(STRIPE // K):
        pltpu.sync_copy(rows_v, acc.at[pl.ds(base + i * K, K)])
    rem = STRIPE % K
    if rem:
        pltpu.sync_copy(rows_v.at[pl.ds(0, rem)],
                        acc.at[pl.ds(base + (STRIPE // K) * K, rem)])
    plsc.subcore_barrier()

    def _chunk(c, _):
        ebase = (wid * C + c) * K
        pltpu.sync_copy(src_hbm.at[pl.ds(ebase, K)], src_v)
        pltpu.sync_copy(dst_hbm.at[pl.ds(ebase, K)], dst_v)
        pltpu.sync_copy(w_hbm.at[pl.ds(ebase, K)], w_v)
        pltpu.async_copy(x_hbm.at[src_v], rows_v, sem).wait()

        def _grp(g, _):
            w16 = w_v[pl.ds(g * 16, 16)]
            ridx = g * 16 + iota16
            cidx = jnp.zeros((16,), jnp.int32)
            for _j in range(D):
                v = plsc.load_gather(rows_v, [ridx, cidx])
                plsc.store_scatter(rows_v, [ridx, cidx], v * w16)
                cidx = cidx + ones16
            return 0
        lax.fori_loop(0, K // 16, _grp, 0)

        pltpu.sync_copy(rows_v, acc.at[dst_v], add=True)
        return 0
    lax.fori_loop(0, C, _chunk, 0)
    plsc.subcore_barrier()

    pbase = cid * N + sid * STRIPE
    pltpu.sync_copy(acc.at[pl.ds(sid * STRIPE, STRIPE)],
                    part_hbm.at[pl.ds(pbase, STRIPE)])


def _make_accum(src_rows):
    mesh = plsc.VectorSubcoreMesh(core_axis_name="c", subcore_axis_name="s")
    return pl.kernel(
        _accum_body,
        out_type=jax.ShapeDtypeStruct((NC * N, D), jnp.float32),
        mesh=mesh,
        scratch_types=[
            pltpu.VMEM_SHARED((N, D), jnp.float32),
            pltpu.VMEM((K,), jnp.int32),
            pltpu.VMEM((K,), jnp.int32),
            pltpu.VMEM((K,), jnp.float32),
            pltpu.VMEM((K, D), jnp.float32),
            pltpu.SemaphoreType.DMA,
        ],
        name=f"ppr_accum_{src_rows}",
    )


def _blend_body(res_hbm, part_hbm, cur_hbm, r_v, p0_v, p1_v, o_v):
    cid = lax.axis_index("c")
    sid = lax.axis_index("s")

    for i in range(STRIPE // BB):
        rb = sid * STRIPE + i * BB
        pltpu.sync_copy(res_hbm.at[pl.ds(rb, BB)], r_v)
        pltpu.sync_copy(part_hbm.at[pl.ds(rb, BB)], p0_v)
        pltpu.sync_copy(part_hbm.at[pl.ds(N + rb, BB)], p1_v)

        def _row(r, _):
            for j in range(D // 16):
                sl = pl.ds(16 * j, 16)
                o_v[r, sl] = 0.5 * (r_v[r, sl] + p0_v[r, sl] + p1_v[r, sl])
            return 0
        lax.fori_loop(0, BB, _row, 0)
        pltpu.sync_copy(o_v, cur_hbm.at[pl.ds(cid * N + rb, BB)])


def _make_blend():
    mesh = plsc.VectorSubcoreMesh(core_axis_name="c", subcore_axis_name="s")
    return pl.kernel(
        _blend_body,
        out_type=jax.ShapeDtypeStruct((NC * N, D), jnp.float32),
        mesh=mesh,
        scratch_types=[
            pltpu.VMEM((BB, D), jnp.float32),
            pltpu.VMEM((BB, D), jnp.float32),
            pltpu.VMEM((BB, D), jnp.float32),
            pltpu.VMEM((BB, D), jnp.float32),
        ],
        name="ppr_blend",
    )


@jax.jit
def kernel(res, edge_index, edge_weight):
    src = edge_index[0]
    dst = edge_index[1]
    pad = EPAD - E
    src_p = jnp.concatenate([src, jnp.zeros((pad,), jnp.int32)])
    dst_p = jnp.concatenate([dst, jnp.zeros((pad,), jnp.int32)])
    w_p = jnp.concatenate([edge_weight, jnp.zeros((pad,), jnp.float32)])
    half = EPAD // 2
    src_p2 = src_p.at[half:].add(N)

    accum0 = _make_accum(N)
    accum1 = _make_accum(2 * N)
    blend = _make_blend()

    part = accum0(res, src_p, dst_p, w_p)
    cur = blend(res, part)
    for _ in range(2):
        part = accum1(cur, src_p2, dst_p, w_p)
        cur = blend(res, part)
    return cur[:N]


# speedup vs baseline: 3.3034x; 3.3034x over previous
import functools

import jax
import jax.numpy as jnp
from jax import lax
from jax.experimental import pallas as pl
from jax.experimental.pallas import tpu as pltpu
from jax.experimental.pallas import tpu_sc as plsc

N = 10000
D = 128
E = 320000
NC = 2
NS = 16
W = NC * NS
K = 128
C = -(-E // (W * K))
EPAD = W * C * K
STRIPE = 640
BB = 128


def _accum_body(x_hbm, src_hbm, dst_hbm, w_hbm, part_hbm,
                acc, src_v, dst_v, w_v, rows_v, sem):
    cid = lax.axis_index("c")
    sid = lax.axis_index("s")
    wid = cid * NS + sid

    iota16 = lax.iota(jnp.int32, 16)
    ones16 = jnp.ones((16,), jnp.int32)
    zeros16 = jnp.zeros((16,), jnp.float32)

    def _zrow(r, _):
        for j in range(D // 16):
            rows_v[r, pl.ds(16 * j, 16)] = zeros16
        return 0
    lax.fori_loop(0, K, _zrow, 0)
    base = jnp.minimum(sid * STRIPE, N - STRIPE)
    for i in range(STRIPE // K):
        pltpu.sync_copy(rows_v, acc.at[pl.ds(base + i * K, K)])
    plsc.subcore_barrier()

    def _chunk(c, _):
        ebase = (wid * C + c) * K
        pltpu.sync_copy(src_hbm.at[pl.ds(ebase, K)], src_v)
        pltpu.sync_copy(dst_hbm.at[pl.ds(ebase, K)], dst_v)
        pltpu.sync_copy(w_hbm.at[pl.ds(ebase, K)], w_v)
        pltpu.async_copy(x_hbm.at[src_v], rows_v, sem).wait()

        def _grp(g, _):
            w16 = w_v[pl.ds(g * 16, 16)]
            for e in range(16):
                k = g * 16 + e
                for j in range(D // 16):
                    sl = pl.ds(16 * j, 16)
                    rows_v[k, sl] = rows_v[k, sl] * w16[e]
            return 0
        lax.fori_loop(0, K // 16, _grp, 0)

        pltpu.sync_copy(rows_v, acc.at[dst_v], add=True)
        return 0
    lax.fori_loop(0, C, _chunk, 0)
    plsc.subcore_barrier()

    pltpu.sync_copy(acc.at[pl.ds(base, STRIPE)],
                    part_hbm.at[pl.ds(cid * N + base, STRIPE)])


def _make_accum(src_rows):
    mesh = plsc.VectorSubcoreMesh(core_axis_name="c", subcore_axis_name="s")
    return pl.kernel(
        _accum_body,
        out_type=jax.ShapeDtypeStruct((NC * N, D), jnp.float32),
        mesh=mesh,
        scratch_types=[
            pltpu.VMEM_SHARED((N, D), jnp.float32),
            pltpu.VMEM((K,), jnp.int32),
            pltpu.VMEM((K,), jnp.int32),
            pltpu.VMEM((K,), jnp.float32),
            pltpu.VMEM((K, D), jnp.float32),
            pltpu.SemaphoreType.DMA,
        ],
        name=f"ppr_accum_{src_rows}",
    )


def _blend_body(res_hbm, part_hbm, cur_hbm, r_v, p0_v, p1_v, o_v):
    cid = lax.axis_index("c")
    sid = lax.axis_index("s")

    base = jnp.minimum(sid * STRIPE, N - STRIPE)
    for i in range(STRIPE // BB):
        rb = base + i * BB
        pltpu.sync_copy(res_hbm.at[pl.ds(rb, BB)], r_v)
        pltpu.sync_copy(part_hbm.at[pl.ds(rb, BB)], p0_v)
        pltpu.sync_copy(part_hbm.at[pl.ds(N + rb, BB)], p1_v)

        def _row(r, _):
            for j in range(D // 16):
                sl = pl.ds(16 * j, 16)
                o_v[r, sl] = 0.5 * (r_v[r, sl] + p0_v[r, sl] + p1_v[r, sl])
            return 0
        lax.fori_loop(0, BB, _row, 0)
        pltpu.sync_copy(o_v, cur_hbm.at[pl.ds(cid * N + rb, BB)])


def _make_blend():
    mesh = plsc.VectorSubcoreMesh(core_axis_name="c", subcore_axis_name="s")
    return pl.kernel(
        _blend_body,
        out_type=jax.ShapeDtypeStruct((NC * N, D), jnp.float32),
        mesh=mesh,
        scratch_types=[
            pltpu.VMEM((BB, D), jnp.float32),
            pltpu.VMEM((BB, D), jnp.float32),
            pltpu.VMEM((BB, D), jnp.float32),
            pltpu.VMEM((BB, D), jnp.float32),
        ],
        name="ppr_blend",
    )


@jax.jit
def kernel(res, edge_index, edge_weight):
    src = edge_index[0]
    dst = edge_index[1]
    pad = EPAD - E
    src_p = jnp.concatenate([src, jnp.zeros((pad,), jnp.int32)])
    dst_p = jnp.concatenate([dst, jnp.zeros((pad,), jnp.int32)])
    w_p = jnp.concatenate([edge_weight, jnp.zeros((pad,), jnp.float32)])
    half = EPAD // 2
    src_p2 = src_p.at[half:].add(N)

    accum0 = _make_accum(N)
    accum1 = _make_accum(2 * N)
    blend = _make_blend()

    part = accum0(res, src_p, dst_p, w_p)
    cur = blend(res, part)
    for _ in range(2):
        part = accum1(cur, src_p2, dst_p, w_p)
        cur = blend(res, part)
    return cur[:N]
